# baseline (speedup 1.0000x reference)
import functools

import jax
import jax.numpy as jnp
from jax import lax
from jax.experimental import pallas as pl
from jax.experimental.pallas import tpu as pltpu
from jax.experimental.pallas import tpu_sc as plsc

_EPS = 1e-5
_CW = 128


def _sc_gather(table, idx3):
    nw, nch, cw = idx3.shape
    d = table.shape[1]
    mesh = plsc.VectorSubcoreMesh(core_axis_name="c", subcore_axis_name="s")

    @functools.partial(
        pl.kernel,
        mesh=mesh,
        out_type=jax.ShapeDtypeStruct((nw * nch * cw, d), jnp.float32),
        scratch_types=[
            pltpu.VMEM((nch, cw), jnp.int32),
            pltpu.VMEM((2, cw, d), jnp.float32),
            pltpu.SemaphoreType.DMA,
            pltpu.SemaphoreType.DMA,
        ],
        compiler_params=pltpu.CompilerParams(use_tc_tiling_on_sc=False),
    )
    def k(table_hbm, idx_hbm, out_hbm, idx_v, rows_v, sem0, sem1):
        wid = lax.axis_index("s") * 2 + lax.axis_index("c")
        base = wid * (nch * cw)
        sems = (sem0, sem1)
        pltpu.sync_copy(idx_hbm.at[wid], idx_v)

        for b in range(2):
            pltpu.async_copy(table_hbm.at[idx_v.at[b]], rows_v.at[b], sems[b])

        def pair(i, _):
            for b in range(2):
                j = 2 * i + b
                pltpu.make_async_copy(
                    table_hbm.at[idx_v.at[j]], rows_v.at[b], sems[b]
                ).wait()
                pltpu.sync_copy(rows_v.at[b],
                                out_hbm.at[pl.ds(base + j * cw, cw)])
                pltpu.async_copy(table_hbm.at[idx_v.at[j + 2]], rows_v.at[b],
                                 sems[b])
            return 0

        lax.fori_loop(0, nch // 2 - 1, pair, 0)

        for b in range(2):
            j = nch - 2 + b
            pltpu.make_async_copy(
                table_hbm.at[idx_v.at[j]], rows_v.at[b], sems[b]
            ).wait()
            pltpu.sync_copy(rows_v.at[b], out_hbm.at[pl.ds(base + j * cw, cw)])

    return k(table, idx3)


def _halfmean(y, n):
    ml = jnp.mean(y[:, :64], axis=-1, keepdims=True)
    mr = jnp.mean(y[:, 64:], axis=-1, keepdims=True)
    return jnp.concatenate(
        [jnp.broadcast_to(ml, (n, 64)), jnp.broadcast_to(mr, (n, 64))], axis=1)


def _ln128(y, g, b, n):
    mu = _halfmean(y, n)
    c = y - mu
    var = _halfmean(c * c, n)
    return c * lax.rsqrt(var + _EPS) * g + b


def _tc_body(x_ref, v_ref, w_ref, bv_ref, tg_ref, tb_ref, vg_ref, vb_ref,
             fg_ref, fb_ref, o_ref):
    x = x_ref[...]
    n = x.shape[0]
    v = v_ref[...]
    vfull = jnp.concatenate(
        [jnp.broadcast_to(v[:, 0:1], (n, 64)),
         jnp.broadcast_to(v[:, 1:2], (n, 64))], axis=1)
    tok = _ln128(x, tg_ref[...], tb_ref[...], n)
    ve = vfull * w_ref[...] + bv_ref[...]
    val = _ln128(ve, vg_ref[...], vb_ref[...], n)
    o_ref[...] = _ln128((tok + val) * 8.0, fg_ref[...], fb_ref[...], n)


def _tc_fused(g2, v2, w2, bv2, tg, tb, vg, vb, fg, fb, rows):
    bl2 = g2.shape[0]
    grid = (bl2 // rows,)
    wspec = pl.BlockSpec((1, 128), lambda i: (0, 0))
    return pl.pallas_call(
        _tc_body,
        grid=grid,
        in_specs=[
            pl.BlockSpec((rows, 128), lambda i: (i, 0)),
            pl.BlockSpec((rows, 2), lambda i: (i, 0)),
            wspec, wspec, wspec, wspec, wspec, wspec, wspec, wspec,
        ],
        out_specs=pl.BlockSpec((rows, 128), lambda i: (i, 0)),
        out_shape=jax.ShapeDtypeStruct((bl2, 128), jnp.float32),
    )(g2, v2, w2, bv2, tg, tb, vg, vb, fg, fb)


def kernel(tokens, values, table, W_val, b_val, tok_g, tok_b, val_g, val_b,
           fin_g, fin_b):
    b, l = tokens.shape
    d = table.shape[1]
    bl = b * l
    nw = 32
    nch = bl // (nw * _CW)

    idx3 = tokens.reshape(nw, nch, _CW).astype(jnp.int32)
    gathered = _sc_gather(table, idx3)

    t2 = lambda a: jnp.tile(a, 2).reshape(1, 128)
    out = _tc_fused(
        gathered.reshape(bl // 2, 128), values.reshape(bl // 2, 2),
        t2(W_val), t2(b_val), t2(tok_g), t2(tok_b), t2(val_g), t2(val_b),
        t2(fin_g), t2(fin_b), rows=2048,
    )
    return (out.reshape(b, l, d), tokens != 0)


# speedup vs baseline: 1.5898x; 1.5898x over previous
import functools

import jax
import jax.numpy as jnp
from jax import lax
from jax.experimental import pallas as pl
from jax.experimental.pallas import tpu as pltpu
from jax.experimental.pallas import tpu_sc as plsc

_EPS = 1e-5
_CW = 128


def _sc_gather(table, idx3):
    nw, nch, cw = idx3.shape
    d = table.shape[1]
    mesh = plsc.VectorSubcoreMesh(core_axis_name="c", subcore_axis_name="s")

    @functools.partial(
        pl.kernel,
        mesh=mesh,
        out_type=jax.ShapeDtypeStruct((nw * nch * cw, d), jnp.float32),
        scratch_types=[
            pltpu.VMEM((nch, cw), jnp.int32),
            pltpu.VMEM((2, cw, d), jnp.float32),
            pltpu.SemaphoreType.DMA,
            pltpu.SemaphoreType.DMA,
        ],
        compiler_params=pltpu.CompilerParams(use_tc_tiling_on_sc=False),
    )
    def k(table_hbm, idx_hbm, out_hbm, idx_v, rows_v, sem0, sem1):
        wid = lax.axis_index("s") * 2 + lax.axis_index("c")
        base = wid * (nch * cw)
        sems = (sem0, sem1)
        pltpu.sync_copy(idx_hbm.at[wid], idx_v)

        for b in range(2):
            pltpu.async_copy(table_hbm.at[idx_v.at[b]], rows_v.at[b], sems[b])

        def pair(i, _):
            for b in range(2):
                j = 2 * i + b
                pltpu.make_async_copy(
                    table_hbm.at[idx_v.at[j]], rows_v.at[b], sems[b]
                ).wait()
                pltpu.sync_copy(rows_v.at[b],
                                out_hbm.at[pl.ds(base + j * cw, cw)])
                pltpu.async_copy(table_hbm.at[idx_v.at[j + 2]], rows_v.at[b],
                                 sems[b])
            return 0

        lax.fori_loop(0, nch // 2 - 1, pair, 0)

        for b in range(2):
            j = nch - 2 + b
            pltpu.make_async_copy(
                table_hbm.at[idx_v.at[j]], rows_v.at[b], sems[b]
            ).wait()
            pltpu.sync_copy(rows_v.at[b], out_hbm.at[pl.ds(base + j * cw, cw)])

    return k(table, idx3)


def _avg_mat():
    i = lax.broadcasted_iota(jnp.int32, (128, 128), 0)
    j = lax.broadcasted_iota(jnp.int32, (128, 128), 1)
    return jnp.where((i // 64) == (j // 64), 1.0 / 64.0, 0.0).astype(jnp.float32)


def _hm(y, m):
    return jax.lax.dot_general(y, m, (((1,), (0,)), ((), ())),
                               preferred_element_type=jnp.float32)


def _ln128(y, g, b, m):
    mu = _hm(y, m)
    q = _hm(y * y, m)
    r = lax.rsqrt(q - mu * mu + _EPS)
    return (y - mu) * r * g + b


def _bcast2(s, n):
    return jnp.concatenate(
        [jnp.broadcast_to(s[:, 0:1], (n, 64)),
         jnp.broadcast_to(s[:, 1:2], (n, 64))], axis=1)


def _tc_body(x_ref, v_ref, w_ref, bv_ref, tg_ref, tb_ref, vg_ref, vb_ref,
             fg_ref, fb_ref, o_ref):
    x = x_ref[...]
    n = x.shape[0]
    v = v_ref[...]
    m = _avg_mat()

    tok = _ln128(x, tg_ref[...], tb_ref[...], m)

    w = w_ref[...]
    bv = bv_ref[...]
    wm = jnp.sum(w[:, :64], axis=-1, keepdims=True) / 64.0
    bm = jnp.sum(bv[:, :64], axis=-1, keepdims=True) / 64.0
    wc = w - wm
    bc = bv - bm
    a2 = jnp.sum(wc[:, :64] * wc[:, :64], axis=-1, keepdims=True) / 64.0
    ab = jnp.sum(wc[:, :64] * bc[:, :64], axis=-1, keepdims=True) / 64.0
    b2 = jnp.sum(bc[:, :64] * bc[:, :64], axis=-1, keepdims=True) / 64.0
    inv = lax.rsqrt(a2 * v * v + 2.0 * ab * v + b2 + _EPS)
    sfull = _bcast2(v * inv, n)
    tfull = _bcast2(inv, n)
    vg = vg_ref[...]
    val = sfull * (wc * vg) + tfull * (bc * vg) + vb_ref[...]

    o_ref[...] = _ln128((tok + val) * 8.0, fg_ref[...], fb_ref[...], m)


def _tc_fused(g2, v2, w2, bv2, tg, tb, vg, vb, fg, fb, rows):
    bl2 = g2.shape[0]
    grid = (bl2 // rows,)
    wspec = pl.BlockSpec((1, 128), lambda i: (0, 0))
    return pl.pallas_call(
        _tc_body,
        grid=grid,
        in_specs=[
            pl.BlockSpec((rows, 128), lambda i: (i, 0)),
            pl.BlockSpec((rows, 2), lambda i: (i, 0)),
            wspec, wspec, wspec, wspec, wspec, wspec, wspec, wspec,
        ],
        out_specs=pl.BlockSpec((rows, 128), lambda i: (i, 0)),
        out_shape=jax.ShapeDtypeStruct((bl2, 128), jnp.float32),
    )(g2, v2, w2, bv2, tg, tb, vg, vb, fg, fb)


def kernel(tokens, values, table, W_val, b_val, tok_g, tok_b, val_g, val_b,
           fin_g, fin_b):
    b, l = tokens.shape
    d = table.shape[1]
    bl = b * l
    nw = 32
    nch = bl // (nw * _CW)

    idx3 = tokens.reshape(nw, nch, _CW).astype(jnp.int32)
    gathered = _sc_gather(table, idx3)

    t2 = lambda a: jnp.tile(a, 2).reshape(1, 128)
    out = _tc_fused(
        gathered.reshape(bl // 2, 128), values.reshape(bl // 2, 2),
        t2(W_val), t2(b_val), t2(tok_g), t2(tok_b), t2(val_g), t2(val_b),
        t2(fin_g), t2(fin_b), rows=2048,
    )
    return (out.reshape(b, l, d), tokens != 0)
